# baseline (speedup 1.0000x reference)
import functools

import jax
import jax.numpy as jnp
from jax import lax
from jax.experimental import pallas as pl
from jax.experimental.pallas import tpu as pltpu
from jax.experimental.pallas import tpu_sc as plsc

_NSEM = 8


def kernel(indices, table):
    (B,) = indices.shape
    V, D = table.shape

    info = plsc.get_sparse_core_info()
    nw = info.num_cores * info.num_subcores
    b_per_w = B // nw

    mesh = plsc.VectorSubcoreMesh(core_axis_name="c", subcore_axis_name="s")

    @functools.partial(
        pl.kernel,
        mesh=mesh,
        out_type=jax.ShapeDtypeStruct((B, D), jnp.float32),
        scratch_types=[
            pltpu.VMEM((b_per_w,), jnp.int32),
            pltpu.VMEM((b_per_w, D), jnp.float32),
            [pltpu.SemaphoreType.DMA] * _NSEM,
        ],
    )
    def _gather(idx_hbm, tab_hbm, out_hbm, idx_v, rows_v, sems):
        wid = lax.axis_index("s") * info.num_cores + lax.axis_index("c")
        base = wid * b_per_w
        pltpu.sync_copy(idx_hbm.at[pl.ds(base, b_per_w)], idx_v)

        for j in range(b_per_w // 16):
            v = idx_v[pl.ds(j * 16, 16)]
            for k in range(16):
                r = j * 16 + k
                pltpu.async_copy(
                    tab_hbm.at[v[k]], rows_v.at[r], sems[r % _NSEM]
                )

        per_sem = b_per_w // _NSEM
        for s in range(_NSEM):
            pltpu.make_async_copy(
                out_hbm.at[pl.ds(base, per_sem)],
                rows_v.at[pl.ds(0, per_sem)],
                sems[s],
            ).wait()

        pltpu.sync_copy(rows_v, out_hbm.at[pl.ds(base, b_per_w)])

    return _gather(indices, table)
